# Initial kernel scaffold; baseline (speedup 1.0000x reference)
import jax
import jax.numpy as jnp
from jax.experimental import pallas as pl


def kernel(v, v_abs):
    raise NotImplementedError("write your pallas kernel here")


# speedup vs baseline: 5850.1110x; 5850.1110x over previous
import functools

import jax
import jax.numpy as jnp
from jax import lax
from jax.experimental import pallas as pl
from jax.experimental.pallas import tpu as pltpu
from jax.experimental.pallas import tpu_sc as plsc

_TH = 0.5
_TAU = 0.1
_N = 1024
_NC = _N // 16
_NW = _N // 32


def _tc_body(xc_ref, xr_ref, vs_ref, condp_ref, rowmax_ref, vout_ref):
    n = _N
    acc = jnp.zeros((n, n), jnp.float32)
    for s in range(8):
        d0 = xc_ref[:, s:s + 1] - xr_ref[s:s + 1, :]
        d1 = xc_ref[:, 8 + s:9 + s] - xr_ref[8 + s:9 + s, :]
        acc = acc + jnp.sqrt(d0 * d0 + d1 * d1)
    dist = acc / 8.0

    br = lax.broadcasted_iota(jnp.int32, (n, n), 0)
    bc = lax.broadcasted_iota(jnp.int32, (n, n), 1)
    condb = (bc < br) & (dist <= _TH)
    rowmax_ref[...] = jnp.max(jnp.where(condb, bc, -1), axis=1, keepdims=True)

    pc = lax.broadcasted_iota(jnp.int32, (n, _NW), 0)
    pw = lax.broadcasted_iota(jnp.int32, (n, _NW), 1)
    same_word = (pc >> 5) == pw
    val = 1 << (pc & 15)
    pack_lo = jnp.where(same_word & ((pc & 31) < 16), val, 0).astype(jnp.float32)
    pack_hi = jnp.where(same_word & ((pc & 31) >= 16), val, 0).astype(jnp.float32)
    condf = condb.astype(jnp.float32)
    lo = jnp.dot(condf, pack_lo, preferred_element_type=jnp.float32)
    hi = jnp.dot(condf, pack_hi, preferred_element_type=jnp.float32)
    condp_ref[...] = lo.astype(jnp.int32) | (hi.astype(jnp.int32) << 16)

    z = (_TH - dist) * (1.0 / _TAU)
    sig = 1.0 / (1.0 + jnp.exp(-z))
    colsum = jnp.sum(sig, axis=0, keepdims=True)
    vs = vs_ref[...]
    vsoft = jnp.dot(vs, sig, preferred_element_type=jnp.float32) / colsum
    vout_ref[...] = (vs - vsoft) + vsoft


_tc_call = pl.pallas_call(
    _tc_body,
    out_shape=[
        jax.ShapeDtypeStruct((_N, _NW), jnp.int32),
        jax.ShapeDtypeStruct((_N, 1), jnp.int32),
        jax.ShapeDtypeStruct((16, _N), jnp.float32),
    ],
)


def _sc_body(condp_hbm, rowmax_hbm, out_hbm,
             condp_v, rowmax_v, label_v, present_v, rank_v):
    wid = lax.axis_index("c") * 16 + lax.axis_index("s")

    @pl.when(wid == 0)
    def _run():
        pltpu.sync_copy(condp_hbm, condp_v)
        pltpu.sync_copy(rowmax_hbm, rowmax_v)
        lanes = lax.iota(jnp.int32, 16)
        zeros = jnp.zeros((16,), jnp.int32)

        def _init(k, carry):
            label_v[pl.ds(k * 16, 16)] = lanes + k * 16
            present_v[pl.ds(k * 16, 16)] = zeros
            return carry

        lax.fori_loop(0, _NC, _init, 0)

        def _do_row(r):
            rsplat = zeros + r
            cksplat = plsc.load_gather(rowmax_v, [rsplat])
            lr = plsc.load_gather(label_v, [rsplat])

            rbase = rsplat * _NW

            def _upd(k, carry):
                lk = label_v[pl.ds(k * 16, 16)]
                w = plsc.load_gather(condp_v, [rbase + (lk >> 5)])
                bit = (w >> (lk & 31)) & 1
                msk = (bit > 0) | (lk == lr)
                label_v[pl.ds(k * 16, 16)] = jnp.where(msk, cksplat, lk)
                return carry

            lax.fori_loop(0, _NC, _upd, 0)

        def _chunk(kk, carry):
            rm = rowmax_v[pl.ds(kk * 16, 16)]
            nhit = jnp.sum((rm >= 0).astype(jnp.int32))

            @pl.when(nhit > 0)
            def _scan_lanes():
                def _lane(j, c2):
                    ck = jnp.sum(jnp.where(lanes == j, rm, 0))

                    @pl.when(ck >= 0)
                    def _hit():
                        _do_row(kk * 16 + j)

                    return c2

                lax.fori_loop(0, 16, _lane, 0)

            return carry

        lax.fori_loop(0, _NC, _chunk, 0)

        ones = zeros + 1

        def _mark(k, carry):
            plsc.store_scatter(present_v, [label_v[pl.ds(k * 16, 16)]], ones)
            return carry

        lax.fori_loop(0, _NC, _mark, 0)

        def _rank(k, carry):
            ch = present_v[pl.ds(k * 16, 16)]
            cs = jnp.cumsum(ch)
            rank_v[pl.ds(k * 16, 16)] = cs + (carry - 1)
            return carry + jnp.sum(ch)

        lax.fori_loop(0, _NC, _rank, 0)

        def _emit(k, carry):
            lk = label_v[pl.ds(k * 16, 16)]
            present_v[pl.ds(k * 16, 16)] = plsc.load_gather(rank_v, [lk])
            return carry

        lax.fori_loop(0, _NC, _emit, 0)
        pltpu.sync_copy(present_v, out_hbm)


@functools.cache
def _sc_call():
    return pl.kernel(
        _sc_body,
        mesh=plsc.VectorSubcoreMesh(core_axis_name="c", subcore_axis_name="s"),
        compiler_params=pltpu.CompilerParams(needs_layout_passes=False),
        out_type=jax.ShapeDtypeStruct((_N,), jnp.int32),
        scratch_types=[
            pltpu.VMEM((_N * _NW,), jnp.int32),
            pltpu.VMEM((_N,), jnp.int32),
            pltpu.VMEM((_N,), jnp.int32),
            pltpu.VMEM((_N,), jnp.int32),
            pltpu.VMEM((_N,), jnp.int32),
        ],
    )


def kernel(v, v_abs):
    x = v_abs.reshape(16, _N)
    vs = v.reshape(16, _N)
    condp, rowmax, vout = _tc_call(x.T, x, vs)
    indices = _sc_call()(condp.reshape(_N * _NW), rowmax.reshape(_N))
    return (vout.reshape(1, 2, 8, _N), indices)


# speedup vs baseline: 6403.1268x; 1.0945x over previous
import functools

import jax
import jax.numpy as jnp
from jax import lax
from jax.experimental import pallas as pl
from jax.experimental.pallas import tpu as pltpu
from jax.experimental.pallas import tpu_sc as plsc

_TH = 0.5
_TAU = 0.1
_N = 1024
_NC = _N // 16
_NW = _N // 32


def _tc_body(xr_ref, vs_ref, condp_ref, rowmax_ref, vout_ref):
    n = _N
    xr = xr_ref[...]
    xc = xr.T
    acc = jnp.zeros((n, n), jnp.float32)
    for s in range(8):
        d0 = xc[:, s:s + 1] - xr[s:s + 1, :]
        d1 = xc[:, 8 + s:9 + s] - xr[8 + s:9 + s, :]
        acc = acc + jnp.sqrt(d0 * d0 + d1 * d1)
    dist = acc / 8.0

    bi = lax.broadcasted_iota(jnp.int32, (n, n), 0)
    br = lax.broadcasted_iota(jnp.int32, (n, n), 1)
    condt = (bi < br) & (dist <= _TH)
    rowmax_ref[...] = jnp.max(jnp.where(condt, bi, -1), axis=0, keepdims=True)

    pw = lax.broadcasted_iota(jnp.int32, (_NW, n), 0)
    pi = lax.broadcasted_iota(jnp.int32, (_NW, n), 1)
    same_word = (pi >> 5) == pw
    val = 1 << (pi & 15)
    pack_lo = jnp.where(same_word & ((pi & 31) < 16), val, 0).astype(jnp.float32)
    pack_hi = jnp.where(same_word & ((pi & 31) >= 16), val, 0).astype(jnp.float32)
    condf = condt.astype(jnp.float32)
    lo = jnp.dot(pack_lo, condf, preferred_element_type=jnp.float32)
    hi = jnp.dot(pack_hi, condf, preferred_element_type=jnp.float32)
    condp_ref[...] = lo.astype(jnp.int32) | (hi.astype(jnp.int32) << 16)

    z = (_TH - dist) * (1.0 / _TAU)
    sig = 1.0 / (1.0 + jnp.exp(-z))
    colsum = jnp.sum(sig, axis=0, keepdims=True)
    vs = vs_ref[...]
    vsoft = jnp.dot(vs, sig, preferred_element_type=jnp.float32) / colsum
    vout_ref[...] = (vs - vsoft) + vsoft


_tc_call = pl.pallas_call(
    _tc_body,
    out_shape=[
        jax.ShapeDtypeStruct((_NW, _N), jnp.int32),
        jax.ShapeDtypeStruct((1, _N), jnp.int32),
        jax.ShapeDtypeStruct((16, _N), jnp.float32),
    ],
)


def _sc_body(condp_hbm, rowmax_hbm, out_hbm,
             condp_v, rowmax_v, label_v, present_v, rank_v):
    wid = lax.axis_index("c") * 16 + lax.axis_index("s")

    @pl.when(wid == 0)
    def _run():
        pltpu.sync_copy(condp_hbm, condp_v)
        pltpu.sync_copy(rowmax_hbm, rowmax_v)
        lanes = lax.iota(jnp.int32, 16)
        zeros = jnp.zeros((16,), jnp.int32)

        def _init(k, carry):
            label_v[pl.ds(k * 16, 16)] = lanes + k * 16
            present_v[pl.ds(k * 16, 16)] = zeros
            return carry

        lax.fori_loop(0, _NC, _init, 0)

        def _do_row(r):
            rsplat = zeros + r
            cksplat = plsc.load_gather(rowmax_v, [rsplat])
            lr = plsc.load_gather(label_v, [rsplat])

            def _upd(k, carry):
                lk = label_v[pl.ds(k * 16, 16)]
                w = plsc.load_gather(condp_v, [((lk >> 5) << 10) + rsplat])
                bit = (w >> (lk & 31)) & 1
                msk = (bit > 0) | (lk == lr)
                label_v[pl.ds(k * 16, 16)] = jnp.where(msk, cksplat, lk)
                return carry

            lax.fori_loop(0, _NC, _upd, 0)

        def _chunk(kk, carry):
            rm = rowmax_v[pl.ds(kk * 16, 16)]
            nhit = jnp.sum((rm >= 0).astype(jnp.int32))

            @pl.when(nhit > 0)
            def _scan_lanes():
                def _lane(j, c2):
                    ck = jnp.sum(jnp.where(lanes == j, rm, 0))

                    @pl.when(ck >= 0)
                    def _hit():
                        _do_row(kk * 16 + j)

                    return c2

                lax.fori_loop(0, 16, _lane, 0)

            return carry + nhit

        total_hits = lax.fori_loop(0, _NC, _chunk, 0)

        @pl.when(total_hits == 0)
        def _identity():
            def _iota(k, carry):
                present_v[pl.ds(k * 16, 16)] = lanes + k * 16
                return carry

            lax.fori_loop(0, _NC, _iota, 0)

        @pl.when(total_hits > 0)
        def _compact():
            ones = zeros + 1

            def _mark(k, carry):
                plsc.store_scatter(
                    present_v, [label_v[pl.ds(k * 16, 16)]], ones)
                return carry

            lax.fori_loop(0, _NC, _mark, 0)

            def _rank(k, carry):
                ch = present_v[pl.ds(k * 16, 16)]
                cs = jnp.cumsum(ch)
                rank_v[pl.ds(k * 16, 16)] = cs + (carry - 1)
                return carry + jnp.sum(ch)

            lax.fori_loop(0, _NC, _rank, 0)

            def _emit(k, carry):
                lk = label_v[pl.ds(k * 16, 16)]
                present_v[pl.ds(k * 16, 16)] = plsc.load_gather(rank_v, [lk])
                return carry

            lax.fori_loop(0, _NC, _emit, 0)

        pltpu.sync_copy(present_v, out_hbm)


@functools.cache
def _sc_call():
    return pl.kernel(
        _sc_body,
        mesh=plsc.VectorSubcoreMesh(core_axis_name="c", subcore_axis_name="s"),
        compiler_params=pltpu.CompilerParams(needs_layout_passes=False),
        out_type=jax.ShapeDtypeStruct((_N,), jnp.int32),
        scratch_types=[
            pltpu.VMEM((_N * _NW,), jnp.int32),
            pltpu.VMEM((_N,), jnp.int32),
            pltpu.VMEM((_N,), jnp.int32),
            pltpu.VMEM((_N,), jnp.int32),
            pltpu.VMEM((_N,), jnp.int32),
        ],
    )


def kernel(v, v_abs):
    x = v_abs.reshape(16, _N)
    vs = v.reshape(16, _N)
    condp, rowmax, vout = _tc_call(x, vs)
    indices = _sc_call()(condp.reshape(_N * _NW), rowmax.reshape(_N))
    return (vout.reshape(1, 2, 8, _N), indices)


# speedup vs baseline: 7396.1423x; 1.1551x over previous
import functools

import jax
import jax.numpy as jnp
from jax import lax
from jax.experimental import pallas as pl
from jax.experimental.pallas import tpu as pltpu
from jax.experimental.pallas import tpu_sc as plsc

_TH = 0.5
_TAU = 0.1
_N = 1024
_NC = _N // 16
_NW = _N // 32


def _tc_body(xr_ref, vs_ref, condp_ref, rowmax_ref, vout_ref, dist_ref):
    n = _N
    nb = 4
    blk = n // nb
    xr = xr_ref[...]
    xc = xr.T
    for bi in range(nb):
        for bj in range(bi + 1):
            acc = jnp.zeros((blk, blk), jnp.float32)
            xci = xc[bi * blk:(bi + 1) * blk, :]
            for s in range(8):
                d0 = xci[:, s:s + 1] - xr[s:s + 1, bj * blk:(bj + 1) * blk]
                d1 = (xci[:, 8 + s:9 + s]
                      - xr[8 + s:9 + s, bj * blk:(bj + 1) * blk])
                acc = acc + jnp.sqrt(d0 * d0 + d1 * d1)
            dist_ref[bi * blk:(bi + 1) * blk, bj * blk:(bj + 1) * blk] = acc
            if bi != bj:
                dist_ref[bj * blk:(bj + 1) * blk,
                         bi * blk:(bi + 1) * blk] = acc.T
    dist = dist_ref[...] / 8.0

    bi = lax.broadcasted_iota(jnp.int32, (n, n), 0)
    br = lax.broadcasted_iota(jnp.int32, (n, n), 1)
    condt = (bi < br) & (dist <= _TH)
    rowmax_ref[...] = jnp.max(jnp.where(condt, bi, -1), axis=0, keepdims=True)

    pw = lax.broadcasted_iota(jnp.int32, (_NW, n), 0)
    pi = lax.broadcasted_iota(jnp.int32, (_NW, n), 1)
    same_word = (pi >> 5) == pw
    val = 1 << (pi & 15)
    pack_lo = jnp.where(same_word & ((pi & 31) < 16), val, 0).astype(jnp.float32)
    pack_hi = jnp.where(same_word & ((pi & 31) >= 16), val, 0).astype(jnp.float32)
    condf = condt.astype(jnp.float32)
    lo = jnp.dot(pack_lo, condf, preferred_element_type=jnp.float32)
    hi = jnp.dot(pack_hi, condf, preferred_element_type=jnp.float32)
    condp_ref[...] = lo.astype(jnp.int32) | (hi.astype(jnp.int32) << 16)

    z = (_TH - dist) * (1.0 / _TAU)
    sig = 1.0 / (1.0 + jnp.exp(-z))
    colsum = jnp.sum(sig, axis=0, keepdims=True)
    vs = vs_ref[...]
    vsoft = jnp.dot(vs, sig, preferred_element_type=jnp.float32) / colsum
    vout_ref[...] = (vs - vsoft) + vsoft


_tc_call = pl.pallas_call(
    _tc_body,
    out_shape=[
        jax.ShapeDtypeStruct((_NW, _N), jnp.int32),
        jax.ShapeDtypeStruct((1, _N), jnp.int32),
        jax.ShapeDtypeStruct((16, _N), jnp.float32),
    ],
    scratch_shapes=[pltpu.VMEM((_N, _N), jnp.float32)],
)


def _sc_body(condp_hbm, rowmax_hbm, out_hbm,
             condp_v, rowmax_v, label_v, present_v, rank_v):
    wid = lax.axis_index("c") * 16 + lax.axis_index("s")

    @pl.when(wid == 0)
    def _run():
        pltpu.sync_copy(condp_hbm, condp_v)
        pltpu.sync_copy(rowmax_hbm, rowmax_v)
        lanes = lax.iota(jnp.int32, 16)
        zeros = jnp.zeros((16,), jnp.int32)

        def _init(k, carry):
            label_v[pl.ds(k * 16, 16)] = lanes + k * 16
            present_v[pl.ds(k * 16, 16)] = zeros
            return carry

        lax.fori_loop(0, _NC, _init, 0)

        def _do_row(r):
            rsplat = zeros + r
            cksplat = plsc.load_gather(rowmax_v, [rsplat])
            lr = plsc.load_gather(label_v, [rsplat])

            def _upd(k, carry):
                lk = label_v[pl.ds(k * 16, 16)]
                w = plsc.load_gather(condp_v, [((lk >> 5) << 10) + rsplat])
                bit = (w >> (lk & 31)) & 1
                msk = (bit > 0) | (lk == lr)
                label_v[pl.ds(k * 16, 16)] = jnp.where(msk, cksplat, lk)
                return carry

            lax.fori_loop(0, _NC, _upd, 0)

        def _chunk(kk, carry):
            rm = rowmax_v[pl.ds(kk * 16, 16)]
            nhit = jnp.sum((rm >= 0).astype(jnp.int32))

            @pl.when(nhit > 0)
            def _scan_lanes():
                def _lane(j, c2):
                    ck = jnp.sum(jnp.where(lanes == j, rm, 0))

                    @pl.when(ck >= 0)
                    def _hit():
                        _do_row(kk * 16 + j)

                    return c2

                lax.fori_loop(0, 16, _lane, 0)

            return carry + nhit

        total_hits = lax.fori_loop(0, _NC, _chunk, 0)

        @pl.when(total_hits == 0)
        def _identity():
            def _iota(k, carry):
                present_v[pl.ds(k * 16, 16)] = lanes + k * 16
                return carry

            lax.fori_loop(0, _NC, _iota, 0)

        @pl.when(total_hits > 0)
        def _compact():
            ones = zeros + 1

            def _mark(k, carry):
                plsc.store_scatter(
                    present_v, [label_v[pl.ds(k * 16, 16)]], ones)
                return carry

            lax.fori_loop(0, _NC, _mark, 0)

            def _rank(k, carry):
                ch = present_v[pl.ds(k * 16, 16)]
                cs = jnp.cumsum(ch)
                rank_v[pl.ds(k * 16, 16)] = cs + (carry - 1)
                return carry + jnp.sum(ch)

            lax.fori_loop(0, _NC, _rank, 0)

            def _emit(k, carry):
                lk = label_v[pl.ds(k * 16, 16)]
                present_v[pl.ds(k * 16, 16)] = plsc.load_gather(rank_v, [lk])
                return carry

            lax.fori_loop(0, _NC, _emit, 0)

        pltpu.sync_copy(present_v, out_hbm)


@functools.cache
def _sc_call():
    return pl.kernel(
        _sc_body,
        mesh=plsc.VectorSubcoreMesh(core_axis_name="c", subcore_axis_name="s"),
        compiler_params=pltpu.CompilerParams(needs_layout_passes=False),
        out_type=jax.ShapeDtypeStruct((_N,), jnp.int32),
        scratch_types=[
            pltpu.VMEM((_N * _NW,), jnp.int32),
            pltpu.VMEM((_N,), jnp.int32),
            pltpu.VMEM((_N,), jnp.int32),
            pltpu.VMEM((_N,), jnp.int32),
            pltpu.VMEM((_N,), jnp.int32),
        ],
    )


def kernel(v, v_abs):
    x = v_abs.reshape(16, _N)
    vs = v.reshape(16, _N)
    condp, rowmax, vout = _tc_call(x, vs)
    indices = _sc_call()(condp.reshape(_N * _NW), rowmax.reshape(_N))
    return (vout.reshape(1, 2, 8, _N), indices)


# speedup vs baseline: 8156.4589x; 1.1028x over previous
import functools

import jax
import jax.numpy as jnp
from jax import lax
from jax.experimental import pallas as pl
from jax.experimental.pallas import tpu as pltpu
from jax.experimental.pallas import tpu_sc as plsc

_TH = 0.5
_TAU = 0.1
_N = 1024
_NC = _N // 16
_NW = _N // 32


def _tc_body(xr_ref, vs_ref, condp_ref, rowmax_ref, vout_ref, dist_ref):
    n = _N
    nb = 4
    blk = n // nb
    xr = xr_ref[...]
    xc = xr.T
    for bi in range(nb):
        for bj in range(bi + 1):
            acc = jnp.zeros((blk, blk), jnp.float32)
            xci = xc[bi * blk:(bi + 1) * blk, :]
            for s in range(8):
                d0 = xci[:, s:s + 1] - xr[s:s + 1, bj * blk:(bj + 1) * blk]
                d1 = (xci[:, 8 + s:9 + s]
                      - xr[8 + s:9 + s, bj * blk:(bj + 1) * blk])
                acc = acc + jnp.sqrt(d0 * d0 + d1 * d1)
            dist_ref[bi * blk:(bi + 1) * blk, bj * blk:(bj + 1) * blk] = acc
            if bi != bj:
                dist_ref[bj * blk:(bj + 1) * blk,
                         bi * blk:(bi + 1) * blk] = acc.T
    dist = dist_ref[...] / 8.0

    bi = lax.broadcasted_iota(jnp.int32, (n, n), 0)
    br = lax.broadcasted_iota(jnp.int32, (n, n), 1)
    condt = (bi < br) & (dist <= _TH)
    rowmax_ref[...] = jnp.max(jnp.where(condt, bi, -1), axis=0, keepdims=True)

    pw = lax.broadcasted_iota(jnp.int32, (_NW, n), 0)
    pi = lax.broadcasted_iota(jnp.int32, (_NW, n), 1)
    same_word = (pi >> 5) == pw
    val = 1 << (pi & 15)
    pack_lo = jnp.where(same_word & ((pi & 31) < 16), val, 0).astype(jnp.float32)
    pack_hi = jnp.where(same_word & ((pi & 31) >= 16), val, 0).astype(jnp.float32)
    condf = condt.astype(jnp.float32)
    lo = jnp.dot(pack_lo, condf, preferred_element_type=jnp.float32)
    hi = jnp.dot(pack_hi, condf, preferred_element_type=jnp.float32)
    condp_ref[...] = lo.astype(jnp.int32) | (hi.astype(jnp.int32) << 16)

    z = (_TH - dist) * (1.0 / _TAU)
    sig = 1.0 / (1.0 + jnp.exp(-z))
    colsum = jnp.sum(sig, axis=0, keepdims=True)
    vs = vs_ref[...]
    vsoft = jnp.dot(vs, sig, preferred_element_type=jnp.float32) / colsum
    vout_ref[...] = (vs - vsoft) + vsoft


_tc_call = pl.pallas_call(
    _tc_body,
    out_shape=[
        jax.ShapeDtypeStruct((_NW, _N), jnp.int32),
        jax.ShapeDtypeStruct((1, _N), jnp.int32),
        jax.ShapeDtypeStruct((16, _N), jnp.float32),
    ],
    scratch_shapes=[pltpu.VMEM((_N, _N), jnp.float32)],
)


def _sc_body(condp_hbm, rowmax_hbm, out_hbm,
             condp_v, rowmax_v, label_v, present_v, rank_v):
    wid = lax.axis_index("c") * 16 + lax.axis_index("s")

    @pl.when(wid == 0)
    def _run():
        pltpu.sync_copy(rowmax_hbm, rowmax_v)
        lanes = lax.iota(jnp.int32, 16)
        zeros = jnp.zeros((16,), jnp.int32)

        def _count(k, acc):
            rm = rowmax_v[pl.ds(k * 16, 16)]
            return acc + (rm >= 0).astype(jnp.int32)

        total_hits = jnp.sum(lax.fori_loop(0, _NC, _count, zeros))

        @pl.when(total_hits == 0)
        def _identity():
            def _iota(k, carry):
                present_v[pl.ds(k * 16, 16)] = lanes + k * 16
                return carry

            lax.fori_loop(0, _NC, _iota, 0)

        @pl.when(total_hits > 0)
        def _cluster():
            pltpu.sync_copy(condp_hbm, condp_v)

            def _init(k, carry):
                label_v[pl.ds(k * 16, 16)] = lanes + k * 16
                present_v[pl.ds(k * 16, 16)] = zeros
                return carry

            lax.fori_loop(0, _NC, _init, 0)

            _cluster_and_compact(condp_v, rowmax_v, label_v, present_v,
                                 rank_v, lanes, zeros)

        pltpu.sync_copy(present_v, out_hbm)


def _cluster_and_compact(condp_v, rowmax_v, label_v, present_v, rank_v,
                         lanes, zeros):
        def _do_row(r):
            rsplat = zeros + r
            cksplat = plsc.load_gather(rowmax_v, [rsplat])
            lr = plsc.load_gather(label_v, [rsplat])

            def _upd(k, carry):
                lk = label_v[pl.ds(k * 16, 16)]
                w = plsc.load_gather(condp_v, [((lk >> 5) << 10) + rsplat])
                bit = (w >> (lk & 31)) & 1
                msk = (bit > 0) | (lk == lr)
                label_v[pl.ds(k * 16, 16)] = jnp.where(msk, cksplat, lk)
                return carry

            lax.fori_loop(0, _NC, _upd, 0)

        def _chunk(kk, carry):
            rm = rowmax_v[pl.ds(kk * 16, 16)]
            nhit = jnp.sum((rm >= 0).astype(jnp.int32))

            @pl.when(nhit > 0)
            def _scan_lanes():
                def _lane(j, c2):
                    ck = jnp.sum(jnp.where(lanes == j, rm, 0))

                    @pl.when(ck >= 0)
                    def _hit():
                        _do_row(kk * 16 + j)

                    return c2

                lax.fori_loop(0, 16, _lane, 0)

            return carry

        lax.fori_loop(0, _NC, _chunk, 0)

        ones = zeros + 1

        def _mark(k, carry):
            plsc.store_scatter(present_v, [label_v[pl.ds(k * 16, 16)]], ones)
            return carry

        lax.fori_loop(0, _NC, _mark, 0)

        def _rank(k, carry):
            ch = present_v[pl.ds(k * 16, 16)]
            cs = jnp.cumsum(ch)
            rank_v[pl.ds(k * 16, 16)] = cs + (carry - 1)
            return carry + jnp.sum(ch)

        lax.fori_loop(0, _NC, _rank, 0)

        def _emit(k, carry):
            lk = label_v[pl.ds(k * 16, 16)]
            present_v[pl.ds(k * 16, 16)] = plsc.load_gather(rank_v, [lk])
            return carry

        lax.fori_loop(0, _NC, _emit, 0)


@functools.cache
def _sc_call():
    return pl.kernel(
        _sc_body,
        mesh=plsc.VectorSubcoreMesh(core_axis_name="c", subcore_axis_name="s"),
        compiler_params=pltpu.CompilerParams(needs_layout_passes=False),
        out_type=jax.ShapeDtypeStruct((_N,), jnp.int32),
        scratch_types=[
            pltpu.VMEM((_N * _NW,), jnp.int32),
            pltpu.VMEM((_N,), jnp.int32),
            pltpu.VMEM((_N,), jnp.int32),
            pltpu.VMEM((_N,), jnp.int32),
            pltpu.VMEM((_N,), jnp.int32),
        ],
    )


def kernel(v, v_abs):
    x = v_abs.reshape(16, _N)
    vs = v.reshape(16, _N)
    condp, rowmax, vout = _tc_call(x, vs)
    indices = _sc_call()(condp.reshape(_N * _NW), rowmax.reshape(_N))
    return (vout.reshape(1, 2, 8, _N), indices)
